# Initial kernel scaffold; baseline (speedup 1.0000x reference)
import jax
import jax.numpy as jnp
from jax.experimental import pallas as pl


def kernel(tokens, token_table, pos_table):
    raise NotImplementedError("write your pallas kernel here")


# speedup vs baseline: 3.7583x; 3.7583x over previous
import functools

import jax
import jax.numpy as jnp
from jax import lax
from jax.experimental import pallas as pl
from jax.experimental.pallas import tpu as pltpu
from jax.experimental.pallas import tpu_sc as plsc

_BATCH = 4096
_SEQ = 200
_EMBED = 64
_NC = 2
_NS = 16
_NW = _NC * _NS
_ROWS_PER_W = _BATCH // _NW
_CHUNK_ROWS = 4
_CHUNK = _CHUNK_ROWS * _SEQ
_NCHUNKS = _ROWS_PER_W // _CHUNK_ROWS


def _sc_embed(tokens_flat, token_table, pos_tiled):
    mesh = plsc.VectorSubcoreMesh(
        core_axis_name="c", subcore_axis_name="s",
        num_cores=_NC, num_subcores=_NS)

    @functools.partial(
        pl.kernel,
        out_type=jax.ShapeDtypeStruct((_BATCH * _SEQ, _EMBED), jnp.float32),
        mesh=mesh,
        compiler_params=pltpu.CompilerParams(use_tc_tiling_on_sc=False),
        scratch_types=[
            pltpu.VMEM((_CHUNK,), jnp.int32),
            pltpu.VMEM((_CHUNK, _EMBED), jnp.float32),
            pltpu.VMEM_SHARED((_CHUNK, _EMBED), jnp.float32),
            pltpu.SemaphoreType.DMA,
        ],
    )
    def k(tok_hbm, table_hbm, pos_hbm, out_hbm, idx_v, rows_v, pos_sh, sem):
        cid = lax.axis_index("c")
        sid = lax.axis_index("s")
        wid = sid * _NC + cid

        @pl.when(sid == 0)
        def _():
            pltpu.sync_copy(pos_hbm, pos_sh)
        plsc.subcore_barrier()

        slab = wid * (_ROWS_PER_W * _SEQ)

        def body(g, carry):
            base = pl.multiple_of(slab + g * _CHUNK, 8)
            pltpu.sync_copy(tok_hbm.at[pl.ds(base, _CHUNK)], idx_v)
            pltpu.sync_copy(pos_sh, rows_v)
            pltpu.async_copy(table_hbm.at[idx_v], rows_v, sem, add=True).wait()
            pltpu.sync_copy(rows_v, out_hbm.at[pl.ds(base, _CHUNK)])
            return carry

        lax.fori_loop(0, _NCHUNKS, body, 0)

    return k(tokens_flat, token_table, pos_tiled)


def kernel(tokens, token_table, pos_table):
    tokens_flat = tokens.reshape(-1).astype(jnp.int32)
    pos_tiled = jnp.tile(pos_table, (_CHUNK_ROWS, 1))
    out = _sc_embed(tokens_flat, token_table, pos_tiled)
    return out.reshape(_BATCH, _SEQ, _EMBED)


# speedup vs baseline: 4.0451x; 1.0763x over previous
import functools

import jax
import jax.numpy as jnp
from jax import lax
from jax.experimental import pallas as pl
from jax.experimental.pallas import tpu as pltpu
from jax.experimental.pallas import tpu_sc as plsc

_BATCH = 4096
_SEQ = 200
_EMBED = 64
_NC = 2
_NS = 16
_NW = _NC * _NS
_ROWS_PER_W = _BATCH // _NW
_CHUNK_ROWS = 4
_CHUNK = _CHUNK_ROWS * _SEQ
_NCHUNKS = _ROWS_PER_W // _CHUNK_ROWS


def _sc_embed(tokens_flat, token_table, pos_tiled):
    mesh = plsc.VectorSubcoreMesh(
        core_axis_name="c", subcore_axis_name="s",
        num_cores=_NC, num_subcores=_NS)

    @functools.partial(
        pl.kernel,
        out_type=jax.ShapeDtypeStruct((_BATCH * _SEQ, _EMBED), jnp.float32),
        mesh=mesh,
        compiler_params=pltpu.CompilerParams(use_tc_tiling_on_sc=False),
        scratch_types=[
            [pltpu.VMEM((_CHUNK,), jnp.int32)] * 2,
            [pltpu.VMEM((_CHUNK, _EMBED), jnp.float32)] * 2,
            pltpu.VMEM_SHARED((_CHUNK, _EMBED), jnp.float32),
            [pltpu.SemaphoreType.DMA] * 2,
            [pltpu.SemaphoreType.DMA] * 2,
            [pltpu.SemaphoreType.DMA] * 2,
            [pltpu.SemaphoreType.DMA] * 2,
        ],
    )
    def k(tok_hbm, table_hbm, pos_hbm, out_hbm, idx_v, rows_v, pos_sh,
          sem_idx, sem_pos, sem_g, sem_out):
        cid = lax.axis_index("c")
        sid = lax.axis_index("s")
        wid = sid * _NC + cid

        @pl.when(sid == 0)
        def _():
            pltpu.sync_copy(pos_hbm, pos_sh)
        plsc.subcore_barrier()

        slab = wid * (_ROWS_PER_W * _SEQ)

        def chunk_base(g):
            return pl.multiple_of(slab + g * _CHUNK, 8)

        for p in range(2):
            base = chunk_base(p)
            pltpu.async_copy(tok_hbm.at[pl.ds(base, _CHUNK)], idx_v[p],
                             sem_idx[p])
            pltpu.async_copy(pos_sh, rows_v[p], sem_pos[p])

        def body(i, carry):
            for p in range(2):
                g = i * 2 + p
                base = chunk_base(g)
                pltpu.make_async_copy(tok_hbm.at[pl.ds(base, _CHUNK)],
                                      idx_v[p], sem_idx[p]).wait()
                pltpu.make_async_copy(pos_sh, rows_v[p], sem_pos[p]).wait()
                pltpu.async_copy(table_hbm.at[idx_v[p]], rows_v[p], sem_g[p],
                                 add=True).wait()
                pltpu.async_copy(rows_v[p], out_hbm.at[pl.ds(base, _CHUNK)],
                                 sem_out[p])

                @pl.when(g + 2 < _NCHUNKS)
                def _():
                    nbase = chunk_base(g + 2)
                    pltpu.make_async_copy(
                        rows_v[p], out_hbm.at[pl.ds(base, _CHUNK)],
                        sem_out[p]).wait()
                    pltpu.async_copy(tok_hbm.at[pl.ds(nbase, _CHUNK)],
                                     idx_v[p], sem_idx[p])
                    pltpu.async_copy(pos_sh, rows_v[p], sem_pos[p])
            return carry

        lax.fori_loop(0, _NCHUNKS // 2, body, 0)
        for p in range(2):
            g = _NCHUNKS - 2 + p
            base = chunk_base(g)
            pltpu.make_async_copy(rows_v[p], out_hbm.at[pl.ds(base, _CHUNK)],
                                  sem_out[p]).wait()

    return k(tokens_flat, token_table, pos_tiled)


def kernel(tokens, token_table, pos_table):
    tokens_flat = tokens.reshape(-1).astype(jnp.int32)
    pos_tiled = jnp.tile(pos_table, (_CHUNK_ROWS, 1))
    out = _sc_embed(tokens_flat, token_table, pos_tiled)
    return out.reshape(_BATCH, _SEQ, _EMBED)
